# baseline (speedup 1.0000x reference)
import functools

import numpy as np

import jax
import jax.numpy as jnp
from jax import lax
from jax.experimental import pallas as pl
from jax.experimental.pallas import tpu as pltpu
from jax.experimental.pallas import tpu_sc as plsc

B = 5
M = 2000
N_NODES = B * M
NUM_REL = 500
F_IN = 128
F_OUT = 128
NUM_FACT = 320000

NC = 2
NS = 16
LANES = 16

BK = 2048
NBLK = 10
SHARD = BK * NBLK
F_PAD = SHARD * NS
ROWS = BK // 128
SROWS = 2 * ROWS

NKP = 4
QNODES = N_NODES // 4
HALF_NODES = 2 * QNODES
PLANE_Q = QNODES * 128
PLANE_H = 2 * PLANE_Q
QWORDS = NKP * PLANE_Q
GARB = 2048
SPM_WORDS = QWORDS + GARB
ZCHUNK = 5008
ZPER = SPM_WORDS // (ZCHUNK * NS)
TSLICE = PLANE_Q // NS
CCHUNK = 5000
NCHUNK = NKP * TSLICE // CCHUNK

VREGS_PER_ROW = 128 // LANES

_NPAD = F_PAD - NUM_FACT
_PAD_NODE = np.arange(_NPAD, dtype=np.int32) % N_NODES
_PAD_REL = np.arange(_NPAD, dtype=np.int32) % NUM_REL
_PAD_W = np.zeros((_NPAD,), np.float32)


def _make_hist_body(p):
    def _hist_body(heads, tails, rels, wts, out,
                   hbufa, tbufa, rbufa, wbufa, idxba, wuba,
                   hbufb, tbufb, rbufb, wbufb, idxbb, wubb,
                   zbuf, cbufa, cbufb, shared,
                   sem_in, sem_sc, sem_cpg, sem_cps):
        c = lax.axis_index("c")
        s = lax.axis_index("s")
        seta = (hbufa, tbufa, rbufa, wbufa, idxba, wuba)
        setb = (hbufb, tbufb, rbufb, wbufb, idxbb, wubb)

        def zfill(i, _):
            zbuf[pl.ds(i * LANES, LANES)] = jnp.zeros((LANES,), jnp.float32)
            return 0
        lax.fori_loop(0, ZCHUNK // LANES, zfill, 0)

        q = 2 * p + c
        n0 = q * QNODES

        def fire_stage(bi, bufset):
            hb, tb, rb, wb = bufset[:4]
            fb = s * SHARD + bi * BK
            return [
                pltpu.async_copy(heads.at[pl.ds(fb, BK)], hb, sem_in),
                pltpu.async_copy(tails.at[pl.ds(fb, BK)], tb, sem_in),
                pltpu.async_copy(rels.at[pl.ds(fb, BK)], rb, sem_in),
                pltpu.async_copy(wts.at[pl.ds(fb, BK)], wb, sem_in),
            ]

        def drain_stage_dummy():
            for _ in range(4):
                pltpu.make_async_copy(
                    heads.at[pl.ds(0, BK)], hbufa, sem_in).wait()

        def drain_scat_dummy():
            for _ in range(SROWS):
                pltpu.make_async_copy(
                    wts.at[pl.ds(0, 128)],
                    cbufa.at[pl.ds(0, 128)], sem_sc).wait()

        def compute(bufset):
            hb, tb, rb, wb, ib, ub = bufset

            def row(j, _):
                for u in range(VREGS_PER_ROW):
                    off = j * 128 + u * LANES
                    h = hb[pl.ds(off, LANES)]
                    t = tb[pl.ds(off, LANES)]
                    r = rb[pl.ds(off, LANES)]
                    w = wb[pl.ds(off, LANES)]
                    rk = (r >> 7) * PLANE_Q + (r & 127)
                    hn = h - n0
                    tn = t - n0
                    inh = (hn >= 0) & (hn < QNODES)
                    int_ = (tn >= 0) & (tn < QNODES)
                    kh = rk + (hn << 7)
                    kt = rk + (tn << 7)
                    gh = QWORDS + (h & (GARB - 1))
                    gt = QWORDS + (t & (GARB - 1))
                    cs = pl.ds(u * LANES, LANES)
                    ib[j, cs] = jnp.where(inh, kh, gh)
                    ub[j, cs] = jnp.where(inh, w, 0.0)
                    ib[j + ROWS, cs] = jnp.where(int_, kt, gt)
                    ub[j + ROWS, cs] = jnp.where(int_, w, 0.0)
                return 0
            lax.fori_loop(0, ROWS, row, 0)

        def fire_scat(bufset):
            ib, ub = bufset[4], bufset[5]
            return [pltpu.async_copy(
                ub.at[j], shared.at[ib.at[j]], sem_sc, add=True)
                for j in range(SROWS)]

        fire_stage(0, seta)

        zds0 = [pltpu.async_copy(
            zbuf,
            shared.at[pl.ds((s * ZPER + z) * ZCHUNK, ZCHUNK)],
            sem_sc) for z in range(ZPER)]
        for d in zds0:
            d.wait()
        plsc.subcore_barrier()

        def pair(pi, _):
            b0 = 2 * pi
            drain_stage_dummy()

            @pl.when(pi > 0)
            def _():
                drain_scat_dummy()
            compute(seta)
            fire_stage(b0 + 1, setb)
            dsa = fire_scat(seta)
            drain_stage_dummy()
            compute(setb)

            @pl.when(pi < NBLK // 2 - 1)
            def _():
                fire_stage(b0 + 2, seta)
            for d in dsa:
                d.wait()
            fire_scat(setb)
            return 0
        lax.fori_loop(0, NBLK // 2, pair, 0)
        drain_scat_dummy()
        plsc.subcore_barrier()

        bufs = (cbufa, cbufb)
        chunks = [(k * PLANE_Q + s * TSLICE + h * CCHUNK,
                   k * PLANE_H + s * TSLICE + h * CCHUNK)
                  for k in range(NKP) for h in range(TSLICE // CCHUNK)]
        cq = c * PLANE_Q
        dss = [None, None]
        dg = pltpu.async_copy(
            shared.at[pl.ds(chunks[0][0], CCHUNK)], bufs[0], sem_cpg)
        for i in range(NCHUNK):
            bi_ = i % 2
            dg.wait()
            dss[bi_] = pltpu.async_copy(
                bufs[bi_], out.at[pl.ds(cq + chunks[i][1], CCHUNK)],
                sem_cps)
            if i + 1 < NCHUNK:
                nb = (i + 1) % 2
                if dss[nb] is not None:
                    dss[nb].wait()
                    dss[nb] = None
                dg = pltpu.async_copy(
                    shared.at[pl.ds(chunks[i + 1][0], CCHUNK)],
                    bufs[nb], sem_cpg)
        for d in dss:
            if d is not None:
                d.wait()
        plsc.subcore_barrier()
    return _hist_body


def _make_hist(p):
    return functools.partial(
        pl.kernel,
        out_type=jax.ShapeDtypeStruct((NKP * PLANE_H,), jnp.float32),
        mesh=plsc.VectorSubcoreMesh(
            core_axis_name="c", subcore_axis_name="s",
            num_cores=NC, num_subcores=NS),
        scratch_types=(
            [pltpu.VMEM((BK,), jnp.int32),
             pltpu.VMEM((BK,), jnp.int32),
             pltpu.VMEM((BK,), jnp.int32),
             pltpu.VMEM((BK,), jnp.float32),
             pltpu.VMEM((SROWS, 128), jnp.int32),
             pltpu.VMEM((SROWS, 128), jnp.float32)] * 2 +
            [pltpu.VMEM((ZCHUNK,), jnp.float32),
             pltpu.VMEM((CCHUNK,), jnp.float32),
             pltpu.VMEM((CCHUNK,), jnp.float32),
             pltpu.VMEM_SHARED((SPM_WORDS,), jnp.float32),
             pltpu.SemaphoreType.DMA,
             pltpu.SemaphoreType.DMA,
             pltpu.SemaphoreType.DMA,
             pltpu.SemaphoreType.DMA]
        ),
        name="hist_p%d" % p,
    )(_make_hist_body(p))


_hist0 = _make_hist(0)
_hist1 = _make_hist(1)


NODE_BLK = 1000
NBLK_TC = HALF_NODES // NODE_BLK


def _bdot(a, b):
    return jnp.dot(a, b, preferred_element_type=jnp.float32)


def _agg_body(cb0_ref, cb1_ref, cb2_ref, cb3_ref, rf_ref, w_ref, b_ref,
              out_ref, rvh_ref, rvl_ref):
    @pl.when(pl.program_id(0) == 0)
    def _():
        rel = jnp.clip(rf_ref[...], -1000.0, 1000.0)
        rv = jnp.dot(rel, w_ref[...].T,
                     preferred_element_type=jnp.float32,
                     precision=lax.Precision.HIGHEST) + b_ref[...]
        rvh = rv.astype(jnp.bfloat16)
        rvh_ref[...] = rvh
        rvl_ref[...] = (rv - rvh.astype(jnp.float32)).astype(jnp.bfloat16)

    agg = jnp.zeros((NODE_BLK, F_OUT), jnp.float32)
    deg = jnp.zeros((NODE_BLK, 1), jnp.float32)
    for k, cb_ref in enumerate((cb0_ref, cb1_ref, cb2_ref, cb3_ref)):
        cb = cb_ref[...]
        cbh = cb.astype(jnp.bfloat16)
        cbl = (cb - cbh.astype(jnp.float32)).astype(jnp.bfloat16)
        rvh = rvh_ref[k * 128:(k + 1) * 128, :]
        rvl = rvl_ref[k * 128:(k + 1) * 128, :]
        agg = agg + (_bdot(cbh, rvh) + (_bdot(cbl, rvh) + _bdot(cbh, rvl)))
        deg = deg + jnp.sum(cb, axis=1, keepdims=True)
    deg = jnp.maximum(deg, 1.0)
    x = jnp.maximum(agg / deg, 0.0)
    x = jnp.where(jnp.isnan(x), 0.0, x)
    x = jnp.where(x == jnp.inf, 10000.0, x)
    x = jnp.where(x == -jnp.inf, -10000.0, x)
    out_ref[...] = x


def _cb_spec(k):
    return pl.BlockSpec((NODE_BLK, 128), lambda i, k=k: (k * NBLK_TC + i, 0))


_agg = pl.pallas_call(
    _agg_body,
    grid=(NBLK_TC,),
    in_specs=[
        _cb_spec(0), _cb_spec(1), _cb_spec(2), _cb_spec(3),
        pl.BlockSpec((NKP * 128, F_IN), lambda i: (0, 0)),
        pl.BlockSpec((F_OUT, F_IN), lambda i: (0, 0)),
        pl.BlockSpec((1, F_OUT), lambda i: (0, 0)),
    ],
    out_specs=pl.BlockSpec((NODE_BLK, F_OUT), lambda i: (i, 0)),
    out_shape=jax.ShapeDtypeStruct((HALF_NODES, F_OUT), jnp.float32),
    scratch_shapes=[pltpu.VMEM((NKP * 128, F_IN), jnp.bfloat16),
                    pltpu.VMEM((NKP * 128, F_IN), jnp.bfloat16)],
)


def kernel(local_entity, batch_heads, batch_rels, batch_tails, batch_ids,
           fact_ids, weight_list, weight_rel_list, rel_features, W, b):
    heads = jnp.concatenate([batch_heads, _PAD_NODE])
    tails = jnp.concatenate([batch_tails, _PAD_NODE])
    rels = jnp.concatenate([batch_rels, _PAD_REL])
    wts = jnp.concatenate([weight_rel_list, _PAD_W])

    counts0 = _hist0(heads, tails, rels, wts)
    counts1 = _hist1(heads, tails, rels, wts)
    rf_pad = jnp.concatenate(
        [rel_features, jnp.zeros((NKP * 128 - NUM_REL, F_IN), jnp.float32)])
    bb = b.reshape(1, F_OUT)
    out0 = _agg(counts0.reshape(NKP * HALF_NODES, 128),
                counts0.reshape(NKP * HALF_NODES, 128),
                counts0.reshape(NKP * HALF_NODES, 128),
                counts0.reshape(NKP * HALF_NODES, 128), rf_pad, W, bb)
    out1 = _agg(counts1.reshape(NKP * HALF_NODES, 128),
                counts1.reshape(NKP * HALF_NODES, 128),
                counts1.reshape(NKP * HALF_NODES, 128),
                counts1.reshape(NKP * HALF_NODES, 128), rf_pad, W, bb)
    out = jnp.concatenate([out0, out1])
    return out.reshape(B, M, F_OUT)


# speedup vs baseline: 1.0523x; 1.0523x over previous
import functools

import numpy as np

import jax
import jax.numpy as jnp
from jax import lax
from jax.experimental import pallas as pl
from jax.experimental.pallas import tpu as pltpu
from jax.experimental.pallas import tpu_sc as plsc

B = 5
M = 2000
N_NODES = B * M
NUM_REL = 500
F_IN = 128
F_OUT = 128
NUM_FACT = 320000

NC = 2
NS = 16
LANES = 16

BK = 2048
NBLK = 10
SHARD = BK * NBLK
F_PAD = SHARD * NS
ROWS = BK // 128
SROWS = 2 * ROWS

NKP = 4
QNODES = N_NODES // 4
HALF_NODES = 2 * QNODES
PLANE_Q = QNODES * 128
PLANE_H = 2 * PLANE_Q
QWORDS = NKP * PLANE_Q
GARB = 2048
SPM_WORDS = QWORDS + GARB
ZCHUNK = 5008
ZPER = SPM_WORDS // (ZCHUNK * NS)
TSLICE = PLANE_Q // NS
CCHUNK = 5000
NCHUNK = NKP * TSLICE // CCHUNK

VREGS_PER_ROW = 128 // LANES

_NPAD = F_PAD - NUM_FACT
_PAD_NODE = np.arange(_NPAD, dtype=np.int32) % N_NODES
_PAD_REL = np.arange(_NPAD, dtype=np.int32) % NUM_REL
_PAD_W = np.zeros((_NPAD,), np.float32)


def _make_hist_body(p):
    def _hist_body(heads, tails, rels, wts, out,
                   hbufa, tbufa, rbufa, wbufa, idxba, wuba,
                   hbufb, tbufb, rbufb, wbufb, idxbb, wubb,
                   zbuf, cbufa, cbufb, shared,
                   sem_in, sem_sc, sem_cpg, sem_cps):
        c = lax.axis_index("c")
        s = lax.axis_index("s")
        seta = (hbufa, tbufa, rbufa, wbufa, idxba, wuba)
        setb = (hbufb, tbufb, rbufb, wbufb, idxbb, wubb)

        def zfill(i, _):
            zbuf[pl.ds(i * LANES, LANES)] = jnp.zeros((LANES,), jnp.float32)
            return 0
        lax.fori_loop(0, ZCHUNK // LANES, zfill, 0)

        q = 2 * p + c
        n0 = q * QNODES

        def fire_stage(bi, bufset):
            hb, tb, rb, wb = bufset[:4]
            fb = s * SHARD + bi * BK
            return [
                pltpu.async_copy(heads.at[pl.ds(fb, BK)], hb, sem_in),
                pltpu.async_copy(tails.at[pl.ds(fb, BK)], tb, sem_in),
                pltpu.async_copy(rels.at[pl.ds(fb, BK)], rb, sem_in),
                pltpu.async_copy(wts.at[pl.ds(fb, BK)], wb, sem_in),
            ]

        def drain_stage_dummy():
            for _ in range(4):
                pltpu.make_async_copy(
                    heads.at[pl.ds(0, BK)], hbufa, sem_in).wait()

        def drain_scat_dummy():
            for _ in range(SROWS):
                pltpu.make_async_copy(
                    wts.at[pl.ds(0, 128)],
                    cbufa.at[pl.ds(0, 128)], sem_sc).wait()

        def compute(bufset):
            hb, tb, rb, wb, ib, ub = bufset

            def row(j, _):
                for u in range(VREGS_PER_ROW):
                    off = j * 128 + u * LANES
                    h = hb[pl.ds(off, LANES)]
                    t = tb[pl.ds(off, LANES)]
                    r = rb[pl.ds(off, LANES)]
                    w = wb[pl.ds(off, LANES)]
                    rk = (r >> 7) * PLANE_Q + (r & 127)
                    hn = h - n0
                    tn = t - n0
                    inh = (hn >= 0) & (hn < QNODES)
                    int_ = (tn >= 0) & (tn < QNODES)
                    kh = rk + (hn << 7)
                    kt = rk + (tn << 7)
                    gh = QWORDS + (h & (GARB - 1))
                    gt = QWORDS + (t & (GARB - 1))
                    cs = pl.ds(u * LANES, LANES)
                    ib[j, cs] = jnp.where(inh, kh, gh)
                    ub[j, cs] = jnp.where(inh, w, 0.0)
                    ib[j + ROWS, cs] = jnp.where(int_, kt, gt)
                    ub[j + ROWS, cs] = jnp.where(int_, w, 0.0)
                return 0
            lax.fori_loop(0, ROWS, row, 0)

        def fire_scat(bufset):
            ib, ub = bufset[4], bufset[5]
            return [pltpu.async_copy(
                ub.at[j], shared.at[ib.at[j]], sem_sc, add=True)
                for j in range(SROWS)]

        fire_stage(0, seta)

        zds0 = [pltpu.async_copy(
            zbuf,
            shared.at[pl.ds((s * ZPER + z) * ZCHUNK, ZCHUNK)],
            sem_sc) for z in range(ZPER)]
        for d in zds0:
            d.wait()
        plsc.subcore_barrier()

        def pair(pi, _):
            b0 = 2 * pi
            drain_stage_dummy()

            @pl.when(pi > 0)
            def _():
                drain_scat_dummy()
            compute(seta)
            fire_stage(b0 + 1, setb)
            dsa = fire_scat(seta)
            drain_stage_dummy()
            compute(setb)

            @pl.when(pi < NBLK // 2 - 1)
            def _():
                fire_stage(b0 + 2, seta)
            for d in dsa:
                d.wait()
            fire_scat(setb)
            return 0
        lax.fori_loop(0, NBLK // 2, pair, 0)
        drain_scat_dummy()
        plsc.subcore_barrier()

        bufs = (cbufa, cbufb)
        chunks = [(k * PLANE_Q + s * TSLICE + h * CCHUNK,
                   k * PLANE_H + s * TSLICE + h * CCHUNK)
                  for k in range(NKP) for h in range(TSLICE // CCHUNK)]
        cq = c * PLANE_Q
        dss = [None, None]
        dg = pltpu.async_copy(
            shared.at[pl.ds(chunks[0][0], CCHUNK)], bufs[0], sem_cpg)
        for i in range(NCHUNK):
            bi_ = i % 2
            dg.wait()
            dss[bi_] = pltpu.async_copy(
                bufs[bi_], out.at[pl.ds(cq + chunks[i][1], CCHUNK)],
                sem_cps)
            if i + 1 < NCHUNK:
                nb = (i + 1) % 2
                if dss[nb] is not None:
                    dss[nb].wait()
                    dss[nb] = None
                dg = pltpu.async_copy(
                    shared.at[pl.ds(chunks[i + 1][0], CCHUNK)],
                    bufs[nb], sem_cpg)
        for d in dss:
            if d is not None:
                d.wait()
        plsc.subcore_barrier()
    return _hist_body


def _make_hist(p):
    return functools.partial(
        pl.kernel,
        out_type=jax.ShapeDtypeStruct((NKP * PLANE_H,), jnp.float32),
        mesh=plsc.VectorSubcoreMesh(
            core_axis_name="c", subcore_axis_name="s",
            num_cores=NC, num_subcores=NS),
        scratch_types=(
            [pltpu.VMEM((BK,), jnp.int32),
             pltpu.VMEM((BK,), jnp.int32),
             pltpu.VMEM((BK,), jnp.int32),
             pltpu.VMEM((BK,), jnp.float32),
             pltpu.VMEM((SROWS, 128), jnp.int32),
             pltpu.VMEM((SROWS, 128), jnp.float32)] * 2 +
            [pltpu.VMEM((ZCHUNK,), jnp.float32),
             pltpu.VMEM((CCHUNK,), jnp.float32),
             pltpu.VMEM((CCHUNK,), jnp.float32),
             pltpu.VMEM_SHARED((SPM_WORDS,), jnp.float32),
             pltpu.SemaphoreType.DMA,
             pltpu.SemaphoreType.DMA,
             pltpu.SemaphoreType.DMA,
             pltpu.SemaphoreType.DMA]
        ),
        name="hist_p%d" % p,
    )(_make_hist_body(p))


_hist0 = _make_hist(0)
_hist1 = _make_hist(1)


NODE_BLK = 1000
NBLK_TC = HALF_NODES // NODE_BLK


def _bdot(a, b):
    return jnp.dot(a, b, preferred_element_type=jnp.float32)


def _agg_body(cb0_ref, cb1_ref, cb2_ref, cb3_ref, rf_ref, w_ref, b_ref,
              out_ref, rvh_ref, rvl_ref):
    @pl.when(pl.program_id(0) == 0)
    def _():
        rel = jnp.clip(rf_ref[...], -1000.0, 1000.0)
        rv = jnp.dot(rel, w_ref[...].T,
                     preferred_element_type=jnp.float32,
                     precision=lax.Precision.HIGHEST) + b_ref[...]
        rvh = rv.astype(jnp.bfloat16)
        rvh_ref[...] = rvh
        rvl_ref[...] = (rv - rvh.astype(jnp.float32)).astype(jnp.bfloat16)

    agg = jnp.zeros((NODE_BLK, F_OUT), jnp.float32)
    deg = jnp.zeros((NODE_BLK, 1), jnp.float32)
    for k, cb_ref in enumerate((cb0_ref, cb1_ref, cb2_ref, cb3_ref)):
        cb = cb_ref[...]
        cbh = cb.astype(jnp.bfloat16)
        cbl = (cb - cbh.astype(jnp.float32)).astype(jnp.bfloat16)
        rvh = rvh_ref[k * 128:(k + 1) * 128, :]
        rvl = rvl_ref[k * 128:(k + 1) * 128, :]
        agg = agg + (_bdot(cbh, rvh) + (_bdot(cbl, rvh) + _bdot(cbh, rvl)))
        deg = deg + jnp.sum(cb, axis=1, keepdims=True)
    deg = jnp.maximum(deg, 1.0)
    x = jnp.maximum(agg / deg, 0.0)
    x = jnp.where(jnp.isnan(x), 0.0, x)
    x = jnp.where(x == jnp.inf, 10000.0, x)
    x = jnp.where(x == -jnp.inf, -10000.0, x)
    out_ref[...] = x


def _cb_spec(k):
    return pl.BlockSpec((NODE_BLK, 128), lambda i, k=k: (k * NBLK_TC + i, 0))


_agg0 = pl.pallas_call(
    _agg_body,
    grid=(NBLK_TC,),
    in_specs=[
        _cb_spec(0), _cb_spec(1), _cb_spec(2), _cb_spec(3),
        pl.BlockSpec((NKP * 128, F_IN), lambda i: (0, 0)),
        pl.BlockSpec((F_OUT, F_IN), lambda i: (0, 0)),
        pl.BlockSpec((1, F_OUT), lambda i: (0, 0)),
    ],
    out_specs=pl.BlockSpec((NODE_BLK, F_OUT), lambda i: (i, 0)),
    out_shape=jax.ShapeDtypeStruct((N_NODES, F_OUT), jnp.float32),
    scratch_shapes=[pltpu.VMEM((NKP * 128, F_IN), jnp.bfloat16),
                    pltpu.VMEM((NKP * 128, F_IN), jnp.bfloat16)],
)


def _agg_body1(prev_ref, cb0_ref, cb1_ref, cb2_ref, cb3_ref, rf_ref, w_ref,
               b_ref, out_ref, rvh_ref, rvl_ref):
    del prev_ref
    _agg_body(cb0_ref, cb1_ref, cb2_ref, cb3_ref, rf_ref, w_ref, b_ref,
              out_ref, rvh_ref, rvl_ref)


_agg1 = pl.pallas_call(
    _agg_body1,
    grid=(NBLK_TC,),
    in_specs=[
        pl.BlockSpec((8, F_OUT), lambda i: (0, 0)),
        _cb_spec(0), _cb_spec(1), _cb_spec(2), _cb_spec(3),
        pl.BlockSpec((NKP * 128, F_IN), lambda i: (0, 0)),
        pl.BlockSpec((F_OUT, F_IN), lambda i: (0, 0)),
        pl.BlockSpec((1, F_OUT), lambda i: (0, 0)),
    ],
    out_specs=pl.BlockSpec((NODE_BLK, F_OUT),
                           lambda i: (i + NBLK_TC, 0)),
    out_shape=jax.ShapeDtypeStruct((N_NODES, F_OUT), jnp.float32),
    scratch_shapes=[pltpu.VMEM((NKP * 128, F_IN), jnp.bfloat16),
                    pltpu.VMEM((NKP * 128, F_IN), jnp.bfloat16)],
    input_output_aliases={0: 0},
)


def kernel(local_entity, batch_heads, batch_rels, batch_tails, batch_ids,
           fact_ids, weight_list, weight_rel_list, rel_features, W, b):
    heads = jnp.concatenate([batch_heads, _PAD_NODE])
    tails = jnp.concatenate([batch_tails, _PAD_NODE])
    rels = jnp.concatenate([batch_rels, _PAD_REL])
    wts = jnp.concatenate([weight_rel_list, _PAD_W])

    counts0 = _hist0(heads, tails, rels, wts)
    counts1 = _hist1(heads, tails, rels, wts)
    rf_pad = jnp.concatenate(
        [rel_features, jnp.zeros((NKP * 128 - NUM_REL, F_IN), jnp.float32)])
    bb = b.reshape(1, F_OUT)
    cb0 = counts0.reshape(NKP * HALF_NODES, 128)
    cb1 = counts1.reshape(NKP * HALF_NODES, 128)
    out0 = _agg0(cb0, cb0, cb0, cb0, rf_pad, W, bb)
    out = _agg1(out0, cb1, cb1, cb1, cb1, rf_pad, W, bb)
    return out.reshape(B, M, F_OUT)
